# baseline (speedup 1.0000x reference)
import functools

import jax
import jax.numpy as jnp
from jax import lax
from jax.experimental import pallas as pl
from jax.experimental.pallas import tpu as pltpu
from jax.experimental.pallas import tpu_sc as plsc

_LANES = 16


@functools.cache
def _build_sc_gather(b, l_pad, vocab, d):
    info = plsc.get_sparse_core_info()
    nc, ns = info.num_cores, info.num_subcores
    nw = nc * ns
    assert b % nw == 0
    rows_per_w = b // nw
    assert rows_per_w % 2 == 0 and rows_per_w >= 4
    assert l_pad % 8 == 0 and l_pad <= 128

    mesh = plsc.VectorSubcoreMesh(core_axis_name="c", subcore_axis_name="s")

    @functools.partial(
        pl.kernel,
        mesh=mesh,
        out_type=jax.ShapeDtypeStruct((b, l_pad, d), jnp.float32),
        scratch_types=[
            pltpu.VMEM((rows_per_w, l_pad), jnp.int32),
            pltpu.VMEM((l_pad, d), jnp.float32),
            pltpu.VMEM((l_pad, d), jnp.float32),
            pltpu.SemaphoreType.DMA,
            pltpu.SemaphoreType.DMA,
            pltpu.SemaphoreType.DMA,
            pltpu.SemaphoreType.DMA,
        ],
    )
    def sc_gather(ids_hbm, table_hbm, emb_out,
                  idx_v, buf_a, buf_b, gsa, gsb, ssa, ssb):
        wid = lax.axis_index("s") * nc + lax.axis_index("c")
        base = wid * rows_per_w

        pltpu.sync_copy(ids_hbm.at[wid], idx_v)

        def gather(j, buf, sem):
            return pltpu.make_async_copy(table_hbm.at[idx_v.at[j]], buf, sem)

        def scatter(j, buf, sem):
            return pltpu.make_async_copy(buf, emb_out.at[base + j], sem)

        gather(0, buf_a, gsa).start()
        gather(0, buf_a, gsa).wait()
        gather(1, buf_b, gsb).start()
        scatter(0, buf_a, ssa).start()

        def pipe(i, carry):
            c = 2 * i + 1
            gather(c, buf_b, gsb).wait()
            scatter(c - 1, buf_a, ssa).wait()
            gather(c + 1, buf_a, gsa).start()
            scatter(c, buf_b, ssb).start()
            gather(c + 1, buf_a, gsa).wait()
            scatter(c, buf_b, ssb).wait()
            gather(c + 2, buf_b, gsb).start()
            scatter(c + 1, buf_a, ssa).start()
            return carry

        lax.fori_loop(0, rows_per_w // 2 - 1, pipe, 0)

        last = rows_per_w - 1
        gather(last, buf_b, gsb).wait()
        scatter(last - 1, buf_a, ssa).wait()
        scatter(last, buf_b, ssb).start()
        scatter(last, buf_b, ssb).wait()

    return sc_gather


@functools.cache
def _build_relayout_ts(b, l, l_pad, d, nb=16):
    inv = 1.0 / float(l - 1)
    assert b % nb == 0

    def body(src_ref, emb_ref, ts_ref):
        emb_ref[...] = src_ref[:, :l, :]
        pos = lax.broadcasted_iota(jnp.int32, (nb, l), 1)
        ts_ref[...] = pos.astype(jnp.float32) * inv

    return pl.pallas_call(
        body,
        grid=(b // nb,),
        in_specs=[pl.BlockSpec((nb, l_pad, d), lambda i: (i, 0, 0))],
        out_specs=[
            pl.BlockSpec((nb, l, d), lambda i: (i, 0, 0)),
            pl.BlockSpec((nb, l), lambda i: (i, 0)),
        ],
        out_shape=[
            jax.ShapeDtypeStruct((b, l, d), jnp.float32),
            jax.ShapeDtypeStruct((b, l), jnp.float32),
        ],
    )


def kernel(input_ids, table):
    b, l = input_ids.shape
    vocab, d = table.shape
    nw = 32
    l_pad = max((l + 7) // 8 * 8, _LANES)
    ids = input_ids.astype(jnp.int32)
    ids_pad = jnp.pad(ids, ((0, 0), (0, l_pad - l))).reshape(nw, b // nw, l_pad)
    staged = _build_sc_gather(b, l_pad, vocab, d)(ids_pad, table)
    emb, ts = _build_relayout_ts(b, l, l_pad, d)(staged)
    return emb, ts


# speedup vs baseline: 1.0016x; 1.0016x over previous
import functools

import jax
import jax.numpy as jnp
from jax import lax
from jax.experimental import pallas as pl
from jax.experimental.pallas import tpu as pltpu
from jax.experimental.pallas import tpu_sc as plsc

_LANES = 16


@functools.cache
def _build_sc_gather(b, l_pad, vocab, d):
    info = plsc.get_sparse_core_info()
    nc, ns = info.num_cores, info.num_subcores
    nw = nc * ns
    assert b % nw == 0
    rows_per_w = b // nw
    assert rows_per_w % 2 == 0 and rows_per_w >= 4
    assert l_pad % 8 == 0 and l_pad <= 128

    mesh = plsc.VectorSubcoreMesh(core_axis_name="c", subcore_axis_name="s")

    @functools.partial(
        pl.kernel,
        mesh=mesh,
        out_type=jax.ShapeDtypeStruct((b * l_pad, d), jnp.float32),
        scratch_types=[
            pltpu.VMEM((rows_per_w, l_pad), jnp.int32),
            pltpu.VMEM((l_pad, d), jnp.float32),
            pltpu.VMEM((l_pad, d), jnp.float32),
            pltpu.SemaphoreType.DMA,
            pltpu.SemaphoreType.DMA,
            pltpu.SemaphoreType.DMA,
            pltpu.SemaphoreType.DMA,
        ],
    )
    def sc_gather(ids_hbm, table_hbm, emb_out,
                  idx_v, buf_a, buf_b, gsa, gsb, ssa, ssb):
        wid = lax.axis_index("s") * nc + lax.axis_index("c")
        base = wid * rows_per_w

        pltpu.sync_copy(ids_hbm.at[wid], idx_v)

        def gather(j, buf, sem):
            return pltpu.make_async_copy(table_hbm.at[idx_v.at[j]], buf, sem)

        def scatter(j, buf, sem):
            dst = emb_out.at[pl.ds((base + j) * l_pad, l_pad)]
            return pltpu.make_async_copy(buf, dst, sem)

        gather(0, buf_a, gsa).start()
        gather(0, buf_a, gsa).wait()
        gather(1, buf_b, gsb).start()
        scatter(0, buf_a, ssa).start()

        def pipe(i, carry):
            c = 2 * i + 1
            gather(c, buf_b, gsb).wait()
            scatter(c - 1, buf_a, ssa).wait()
            gather(c + 1, buf_a, gsa).start()
            scatter(c, buf_b, ssb).start()
            gather(c + 1, buf_a, gsa).wait()
            scatter(c, buf_b, ssb).wait()
            gather(c + 2, buf_b, gsb).start()
            scatter(c + 1, buf_a, ssa).start()
            return carry

        lax.fori_loop(0, rows_per_w // 2 - 1, pipe, 0)

        last = rows_per_w - 1
        gather(last, buf_b, gsb).wait()
        scatter(last - 1, buf_a, ssa).wait()
        scatter(last, buf_b, ssb).start()
        scatter(last, buf_b, ssb).wait()

    return sc_gather


@functools.cache
def _build_relayout_ts(b, l, l_pad, d, nb=16):
    inv = 1.0 / float(l - 1)
    assert b % nb == 0

    def body(src_ref, emb_ref, ts_ref):
        emb_ref[...] = src_ref[...].reshape(nb, l_pad, d)[:, :l, :]
        pos = lax.broadcasted_iota(jnp.int32, (nb, l), 1)
        ts_ref[...] = pos.astype(jnp.float32) * inv

    return pl.pallas_call(
        body,
        grid=(b // nb,),
        in_specs=[pl.BlockSpec((nb * l_pad, d), lambda i: (i, 0))],
        out_specs=[
            pl.BlockSpec((nb, l, d), lambda i: (i, 0, 0)),
            pl.BlockSpec((nb, l), lambda i: (i, 0)),
        ],
        out_shape=[
            jax.ShapeDtypeStruct((b, l, d), jnp.float32),
            jax.ShapeDtypeStruct((b, l), jnp.float32),
        ],
    )


def kernel(input_ids, table):
    b, l = input_ids.shape
    vocab, d = table.shape
    nw = 32
    l_pad = max((l + 7) // 8 * 8, _LANES)
    ids = input_ids.astype(jnp.int32)
    ids_pad = jnp.pad(ids, ((0, 0), (0, l_pad - l))).reshape(nw, b // nw, l_pad)
    staged = _build_sc_gather(b, l_pad, vocab, d)(ids_pad, table)
    emb, ts = _build_relayout_ts(b, l, l_pad, d)(staged)
    return emb, ts


# speedup vs baseline: 2.0687x; 2.0653x over previous
import functools

import jax
import jax.numpy as jnp
from jax import lax
from jax.experimental import pallas as pl
from jax.experimental.pallas import tpu as pltpu
from jax.experimental.pallas import tpu_sc as plsc

_LANES = 16


@functools.cache
def _build_sc_gather(b, l_pad, vocab, d):
    info = plsc.get_sparse_core_info()
    nc, ns = info.num_cores, info.num_subcores
    nw = nc * ns
    assert b % nw == 0
    rows_per_w = b // nw
    assert rows_per_w % 2 == 0 and rows_per_w >= 4
    assert l_pad % 8 == 0 and l_pad <= 128

    mesh = plsc.VectorSubcoreMesh(core_axis_name="c", subcore_axis_name="s")

    @functools.partial(
        pl.kernel,
        mesh=mesh,
        out_type=jax.ShapeDtypeStruct((b * l_pad, d), jnp.float32),
        scratch_types=[
            pltpu.VMEM((rows_per_w, l_pad), jnp.int32),
            pltpu.VMEM((l_pad, d), jnp.float32),
            pltpu.VMEM((l_pad, d), jnp.float32),
            pltpu.SemaphoreType.DMA,
            pltpu.SemaphoreType.DMA,
            pltpu.SemaphoreType.DMA,
            pltpu.SemaphoreType.DMA,
        ],
    )
    def sc_gather(ids_hbm, table_hbm, emb_out,
                  idx_v, buf_a, buf_b, gsa, gsb, ssa, ssb):
        wid = lax.axis_index("s") * nc + lax.axis_index("c")
        base = wid * rows_per_w

        pltpu.sync_copy(ids_hbm.at[wid], idx_v)

        def gather(j, buf, sem):
            return pltpu.make_async_copy(table_hbm.at[idx_v.at[j]], buf, sem)

        def scatter(j, buf, sem):
            dst = emb_out.at[pl.ds((base + j) * l_pad, l_pad)]
            return pltpu.make_async_copy(buf, dst, sem)

        gather(0, buf_a, gsa).start()
        gather(0, buf_a, gsa).wait()
        gather(1, buf_b, gsb).start()
        scatter(0, buf_a, ssa).start()

        def pipe(i, carry):
            c = 2 * i + 1
            gather(c, buf_b, gsb).wait()
            scatter(c - 1, buf_a, ssa).wait()
            gather(c + 1, buf_a, gsa).start()
            scatter(c, buf_b, ssb).start()
            gather(c + 1, buf_a, gsa).wait()
            scatter(c, buf_b, ssb).wait()
            gather(c + 2, buf_b, gsb).start()
            scatter(c + 1, buf_a, ssa).start()
            return carry

        lax.fori_loop(0, rows_per_w // 2 - 1, pipe, 0)

        last = rows_per_w - 1
        gather(last, buf_b, gsb).wait()
        scatter(last - 1, buf_a, ssa).wait()
        scatter(last, buf_b, ssb).start()
        scatter(last, buf_b, ssb).wait()

    return sc_gather


@functools.cache
def _build_relayout_ts(b, l, l_pad, d, nb=16):
    inv = 1.0 / float(l - 1)
    assert b % nb == 0

    def body(src_ref, emb_ref, ts_ref):
        emb_ref[...] = src_ref[...].reshape(nb, l_pad, d)[:, :l, :]
        pos = lax.broadcasted_iota(jnp.int32, (nb, l), 1)
        ts_ref[...] = pos.astype(jnp.float32) * inv

    return pl.pallas_call(
        body,
        grid=(b // nb,),
        in_specs=[pl.BlockSpec((nb * l_pad, d), lambda i: (i, 0))],
        out_specs=[
            pl.BlockSpec((nb, l, d), lambda i: (i, 0, 0)),
            pl.BlockSpec((nb, l), lambda i: (i, 0)),
        ],
        out_shape=[
            jax.ShapeDtypeStruct((b, l, d), jnp.float32),
            jax.ShapeDtypeStruct((b, l), jnp.float32),
        ],
    )


def kernel(input_ids, table):
    b, l = input_ids.shape
    vocab, d = table.shape
    nw = 32
    l_pad = max((l + 7) // 8 * 8, _LANES)
    ids = input_ids.astype(jnp.int32)
    ids_pad = jnp.concatenate([ids, ids[:, : l_pad - l]], axis=1)
    ids_pad = ids_pad.reshape(nw, b // nw, l_pad)
    staged = _build_sc_gather(b, l_pad, vocab, d)(ids_pad, table)
    emb, ts = _build_relayout_ts(b, l, l_pad, d)(staged)
    return emb, ts
